# Initial kernel scaffold; baseline (speedup 1.0000x reference)
import jax
import jax.numpy as jnp
from jax.experimental import pallas as pl


def kernel(input):
    raise NotImplementedError("write your pallas kernel here")


# speedup vs baseline: 9.2510x; 9.2510x over previous
import functools

import jax
import jax.numpy as jnp
from jax import lax
from jax.experimental import pallas as pl
from jax.experimental.pallas import tpu as pltpu
from jax.experimental.pallas import tpu_sc as plsc

N = 524288
C = 32
K = 52429
NW = 32
ROWS_W = N // NW
CHUNK = 1024
NCH = ROWS_W // CHUNK
UNROLL = 4
B = 256
HSIZE = B * C

MIN32 = -2147483648


def _make_round(shift: int, first: bool):
    mesh = plsc.VectorSubcoreMesh(core_axis_name="c", subcore_axis_name="s")
    out_type = (
        jax.ShapeDtypeStruct((NW, HSIZE), jnp.int32),
        jax.ShapeDtypeStruct((NW, HSIZE), jnp.float32),
    )
    scratch = [
        pltpu.VMEM((2, CHUNK * C), jnp.float32),
        pltpu.VMEM((HSIZE,), jnp.int32),
        pltpu.VMEM((HSIZE,), jnp.float32),
        pltpu.VMEM((C,), jnp.int32),
        pltpu.SemaphoreType.DMA,
        pltpu.SemaphoreType.DMA,
    ]

    def body(*refs):
        if first:
            x_hbm, cnt_hbm, sum_hbm, stage, cnt_v, sum_v, pref_v, sem0, sem1 = refs
            pref_hbm = None
        else:
            x_hbm, pref_hbm, cnt_hbm, sum_hbm, stage, cnt_v, sum_v, pref_v, sem0, sem1 = refs

        wid = lax.axis_index("s") * 2 + lax.axis_index("c")
        row0 = wid * ROWS_W

        zi = jnp.zeros((16,), jnp.int32)
        zf = jnp.zeros((16,), jnp.float32)

        def zero_body(i, carry):
            cnt_v[pl.ds(i * 16, 16)] = zi
            sum_v[pl.ds(i * 16, 16)] = zf
            return carry

        lax.fori_loop(0, HSIZE // 16, zero_body, 0)

        if not first:
            pltpu.sync_copy(pref_hbm, pref_v)
            pref_lo = pref_v[pl.ds(0, 16)]
            pref_hi = pref_v[pl.ds(16, 16)]
        else:
            pref_lo = pref_hi = None

        iota = lax.iota(jnp.int32, 16)
        cols = (iota, iota + 16)
        prefs = (pref_lo, pref_hi)
        ones = jnp.ones((16,), jnp.int32)

        def dma(ch, buf, sem):
            return pltpu.make_async_copy(
                x_hbm.at[pl.ds((row0 + ch * CHUNK) * C, CHUNK * C)],
                stage.at[buf], sem)

        dma(0, 0, sem0).start()
        sems = (sem0, sem1)
        for ch in range(NCH):
            buf = ch & 1
            dma(ch, buf, sems[buf]).wait()
            if ch + 1 < NCH:
                dma(ch + 1, 1 - buf, sems[1 - buf]).start()

            def chunk_body(j, carry):
                for u in range(UNROLL):
                    r = j * UNROLL + u
                    for half in range(2):
                        v = stage[buf, pl.ds(r * C + 16 * half, 16)]
                        y = lax.bitcast_convert_type(v, jnp.int32)
                        m = lax.shift_right_arithmetic(y, 31)
                        key = lax.bitwise_xor(
                            y, lax.bitwise_or(m, jnp.int32(MIN32)))
                        bucket = lax.shift_right_logical(key, shift)
                        if shift != 24:
                            bucket = lax.bitwise_and(bucket, jnp.int32(B - 1))
                        idx = bucket * 32 + cols[half]
                        if first:
                            plsc.addupdate_scatter(cnt_v, [idx], ones)
                            plsc.addupdate_scatter(sum_v, [idx], v)
                        else:
                            keyhi = lax.shift_right_logical(key, shift + 8)
                            msk = keyhi == prefs[half]
                            plsc.addupdate_scatter(cnt_v, [idx], ones, mask=msk)
                            plsc.addupdate_scatter(sum_v, [idx], v, mask=msk)
                return carry

            lax.fori_loop(0, CHUNK // UNROLL, chunk_body, 0)

        pltpu.sync_copy(cnt_v, cnt_hbm.at[wid])
        pltpu.sync_copy(sum_v, sum_hbm.at[wid])

    return pl.kernel(
        body, out_type=out_type, mesh=mesh, scratch_types=scratch,
        compiler_params=pltpu.CompilerParams(needs_layout_passes=False))


_ROUNDS = tuple(
    (shift, _make_round(shift, shift == 24)) for shift in (24, 16, 8, 0)
)


def kernel(input):
    x = input.reshape(-1)
    k_rem = jnp.full((C,), K, jnp.int32)
    sum_below = jnp.zeros((C,), jnp.float32)
    prefix = jnp.zeros((C,), jnp.int32)
    for shift, fn in _ROUNDS:
        if shift == 24:
            cnt, sm = fn(x)
        else:
            cnt, sm = fn(x, prefix)
        cntm = cnt.sum(axis=0).reshape(B, C)
        smm = sm.sum(axis=0).reshape(B, C)
        cum = jnp.cumsum(cntm, axis=0)
        b = jnp.argmax(cum >= k_rem[None, :], axis=0).astype(jnp.int32)
        cnt_below = jnp.take_along_axis(cum - cntm, b[None, :], 0)[0]
        smb = jnp.take_along_axis(jnp.cumsum(smm, axis=0) - smm, b[None, :], 0)[0]
        k_rem = k_rem - cnt_below
        sum_below = sum_below + smb
        prefix = prefix * 256 + b
    ybits = jnp.where(prefix < 0, prefix ^ jnp.int32(MIN32), ~prefix)
    tval = lax.bitcast_convert_type(ybits, jnp.float32)
    return -(sum_below + k_rem.astype(jnp.float32) * tval) / jnp.float32(K)


# speedup vs baseline: 22.1293x; 2.3921x over previous
import functools

import jax
import jax.numpy as jnp
from jax import lax
from jax.experimental import pallas as pl
from jax.experimental.pallas import tpu as pltpu
from jax.experimental.pallas import tpu_sc as plsc

N = 524288
C = 32
K = 52429
NW = 32
ROWS_W = N // NW
CHUNK = 1024
NCH = ROWS_W // CHUNK
UNROLL = 4
B = 256
HSIZE = B * C

MIN32 = -2147483648


def _make_round(shift: int, first: bool):
    mesh = plsc.VectorSubcoreMesh(core_axis_name="c", subcore_axis_name="s")
    out_type = (
        jax.ShapeDtypeStruct((NW, HSIZE), jnp.int32),
        jax.ShapeDtypeStruct((NW, HSIZE), jnp.float32),
    )
    scratch = [
        pltpu.VMEM((2, CHUNK * C), jnp.float32),
        pltpu.VMEM((HSIZE,), jnp.int32),
        pltpu.VMEM((HSIZE,), jnp.float32),
        pltpu.VMEM((C,), jnp.int32),
        pltpu.SemaphoreType.DMA,
        pltpu.SemaphoreType.DMA,
    ]

    def body(*refs):
        if first:
            x_hbm, cnt_hbm, sum_hbm, stage, cnt_v, sum_v, pref_v, sem0, sem1 = refs
            pref_hbm = None
        else:
            x_hbm, pref_hbm, cnt_hbm, sum_hbm, stage, cnt_v, sum_v, pref_v, sem0, sem1 = refs

        wid = lax.axis_index("s") * 2 + lax.axis_index("c")
        row0 = wid * ROWS_W

        zi = jnp.zeros((16,), jnp.int32)
        zf = jnp.zeros((16,), jnp.float32)

        def zero_body(i, carry):
            cnt_v[pl.ds(i * 16, 16)] = zi
            sum_v[pl.ds(i * 16, 16)] = zf
            return carry

        lax.fori_loop(0, HSIZE // 16, zero_body, 0)

        if not first:
            pltpu.sync_copy(pref_hbm, pref_v)
            pref_lo = pref_v[pl.ds(0, 16)]
            pref_hi = pref_v[pl.ds(16, 16)]
        else:
            pref_lo = pref_hi = None

        iota = lax.iota(jnp.int32, 16)
        cols = (iota, iota + 16)
        prefs = (pref_lo, pref_hi)
        ones = jnp.ones((16,), jnp.int32)

        def dma(ch, buf, sem):
            return pltpu.make_async_copy(
                x_hbm.at[pl.ds((row0 + ch * CHUNK) * C, CHUNK * C)],
                stage.at[buf], sem)

        dma(0, 0, sem0).start()
        sems = (sem0, sem1)
        for ch in range(NCH):
            buf = ch & 1
            dma(ch, buf, sems[buf]).wait()
            if ch + 1 < NCH:
                dma(ch + 1, 1 - buf, sems[1 - buf]).start()

            def chunk_body(j, carry):
                vs, idxs, msks = [], [], []
                for u in range(UNROLL):
                    r = j * UNROLL + u
                    for half in range(2):
                        vs.append(stage[buf, pl.ds(r * C + 16 * half, 16)])
                for i, v in enumerate(vs):
                    half = i & 1
                    y = lax.bitcast_convert_type(v, jnp.int32)
                    m = lax.shift_right_arithmetic(y, 31)
                    key = lax.bitwise_xor(
                        y, lax.bitwise_or(m, jnp.int32(MIN32)))
                    bucket = lax.shift_right_logical(key, shift)
                    if shift != 24:
                        bucket = lax.bitwise_and(bucket, jnp.int32(B - 1))
                    idxs.append(bucket * 32 + cols[half])
                    if first:
                        msks.append(None)
                    else:
                        keyhi = lax.shift_right_logical(key, shift + 8)
                        msks.append(keyhi == prefs[half])
                for v, idx, msk in zip(vs, idxs, msks):
                    plsc.addupdate_scatter(cnt_v, [idx], ones, mask=msk)
                    plsc.addupdate_scatter(sum_v, [idx], v, mask=msk)
                return carry

            lax.fori_loop(0, CHUNK // UNROLL, chunk_body, 0)

        pltpu.sync_copy(cnt_v, cnt_hbm.at[wid])
        pltpu.sync_copy(sum_v, sum_hbm.at[wid])

    return pl.kernel(
        body, out_type=out_type, mesh=mesh, scratch_types=scratch,
        compiler_params=pltpu.CompilerParams(needs_layout_passes=False))


_ROUNDS = tuple(
    (shift, _make_round(shift, shift == 24)) for shift in (24, 16, 8, 0)
)


def kernel(input):
    x = input.reshape(-1)
    k_rem = jnp.full((C,), K, jnp.int32)
    sum_below = jnp.zeros((C,), jnp.float32)
    prefix = jnp.zeros((C,), jnp.int32)
    for shift, fn in _ROUNDS:
        if shift == 24:
            cnt, sm = fn(x)
        else:
            cnt, sm = fn(x, prefix)
        cntm = cnt.sum(axis=0).reshape(B, C)
        smm = sm.sum(axis=0).reshape(B, C)
        cum = jnp.cumsum(cntm, axis=0)
        b = jnp.argmax(cum >= k_rem[None, :], axis=0).astype(jnp.int32)
        cnt_below = jnp.take_along_axis(cum - cntm, b[None, :], 0)[0]
        smb = jnp.take_along_axis(jnp.cumsum(smm, axis=0) - smm, b[None, :], 0)[0]
        k_rem = k_rem - cnt_below
        sum_below = sum_below + smb
        prefix = prefix * 256 + b
    ybits = jnp.where(prefix < 0, prefix ^ jnp.int32(MIN32), ~prefix)
    tval = lax.bitcast_convert_type(ybits, jnp.float32)
    return -(sum_below + k_rem.astype(jnp.float32) * tval) / jnp.float32(K)


# speedup vs baseline: 25.0025x; 1.1298x over previous
import functools

import jax
import jax.numpy as jnp
from jax import lax
from jax.experimental import pallas as pl
from jax.experimental.pallas import tpu as pltpu
from jax.experimental.pallas import tpu_sc as plsc

N = 524288
C = 32
K = 52429
NW = 32
ROWS_W = N // NW
CHUNK = 1024
NCH = ROWS_W // CHUNK
UNROLL = 4
B = 256
HSIZE = B * C

MIN32 = -2147483648


def _make_round(shift: int, first: bool):
    mesh = plsc.VectorSubcoreMesh(core_axis_name="c", subcore_axis_name="s")
    out_type = (
        jax.ShapeDtypeStruct((NW, HSIZE), jnp.int32),
        jax.ShapeDtypeStruct((NW, HSIZE), jnp.float32),
    )
    scratch = [
        pltpu.VMEM((2, CHUNK, C), jnp.float32),
        pltpu.VMEM((HSIZE,), jnp.int32),
        pltpu.VMEM((HSIZE,), jnp.float32),
        pltpu.VMEM((C,), jnp.int32),
        pltpu.SemaphoreType.DMA,
        pltpu.SemaphoreType.DMA,
    ]

    def body(*refs):
        if first:
            x_hbm, cnt_hbm, sum_hbm, stage, cnt_v, sum_v, pref_v, sem0, sem1 = refs
            pref_hbm = None
        else:
            x_hbm, pref_hbm, cnt_hbm, sum_hbm, stage, cnt_v, sum_v, pref_v, sem0, sem1 = refs

        wid = lax.axis_index("s") * 2 + lax.axis_index("c")
        row0 = wid * ROWS_W

        zi = jnp.zeros((16,), jnp.int32)
        zf = jnp.zeros((16,), jnp.float32)

        def zero_body(i, carry):
            cnt_v[pl.ds(i * 16, 16)] = zi
            sum_v[pl.ds(i * 16, 16)] = zf
            return carry

        lax.fori_loop(0, HSIZE // 16, zero_body, 0)

        if not first:
            pltpu.sync_copy(pref_hbm, pref_v)
            pref_lo = pref_v[pl.ds(0, 16)]
            pref_hi = pref_v[pl.ds(16, 16)]
        else:
            pref_lo = pref_hi = None

        iota = lax.iota(jnp.int32, 16)
        cols = (iota, iota + 16)
        prefs = (pref_lo, pref_hi)
        ones = jnp.ones((16,), jnp.int32)

        def dma(ch, buf, sem):
            return pltpu.make_async_copy(
                x_hbm.at[pl.ds(row0 + ch * CHUNK, CHUNK)],
                stage.at[buf], sem)

        dma(0, 0, sem0).start()
        sems = (sem0, sem1)
        for ch in range(NCH):
            buf = ch & 1
            dma(ch, buf, sems[buf]).wait()
            if ch + 1 < NCH:
                dma(ch + 1, 1 - buf, sems[1 - buf]).start()

            def chunk_body(j, carry):
                vs, idxs, msks = [], [], []
                for u in range(UNROLL):
                    r = j * UNROLL + u
                    for half in range(2):
                        vs.append(stage[buf, r, pl.ds(16 * half, 16)])
                for i, v in enumerate(vs):
                    half = i & 1
                    y = lax.bitcast_convert_type(v, jnp.int32)
                    m = lax.shift_right_arithmetic(y, 31)
                    key = lax.bitwise_xor(
                        y, lax.bitwise_or(m, jnp.int32(MIN32)))
                    bucket = lax.shift_right_logical(key, shift)
                    if shift != 24:
                        bucket = lax.bitwise_and(bucket, jnp.int32(B - 1))
                    idxs.append(bucket * 32 + cols[half])
                    if first:
                        msks.append(None)
                    else:
                        keyhi = lax.shift_right_logical(key, shift + 8)
                        msks.append(keyhi == prefs[half])
                for v, idx, msk in zip(vs, idxs, msks):
                    plsc.addupdate_scatter(cnt_v, [idx], ones, mask=msk)
                    plsc.addupdate_scatter(sum_v, [idx], v, mask=msk)
                return carry

            lax.fori_loop(0, CHUNK // UNROLL, chunk_body, 0)

        pltpu.sync_copy(cnt_v, cnt_hbm.at[wid])
        pltpu.sync_copy(sum_v, sum_hbm.at[wid])

    return pl.kernel(
        body, out_type=out_type, mesh=mesh, scratch_types=scratch,
        compiler_params=pltpu.CompilerParams(
            needs_layout_passes=False, use_tc_tiling_on_sc=False))


_ROUNDS = tuple(
    (shift, _make_round(shift, shift == 24)) for shift in (24, 16, 8, 0)
)


def kernel(input):
    x = input
    k_rem = jnp.full((C,), K, jnp.int32)
    sum_below = jnp.zeros((C,), jnp.float32)
    prefix = jnp.zeros((C,), jnp.int32)
    for shift, fn in _ROUNDS:
        if shift == 24:
            cnt, sm = fn(x)
        else:
            cnt, sm = fn(x, prefix)
        cntm = cnt.sum(axis=0).reshape(B, C)
        smm = sm.sum(axis=0).reshape(B, C)
        cum = jnp.cumsum(cntm, axis=0)
        b = jnp.argmax(cum >= k_rem[None, :], axis=0).astype(jnp.int32)
        cnt_below = jnp.take_along_axis(cum - cntm, b[None, :], 0)[0]
        smb = jnp.take_along_axis(jnp.cumsum(smm, axis=0) - smm, b[None, :], 0)[0]
        k_rem = k_rem - cnt_below
        sum_below = sum_below + smb
        prefix = prefix * 256 + b
    ybits = jnp.where(prefix < 0, prefix ^ jnp.int32(MIN32), ~prefix)
    tval = lax.bitcast_convert_type(ybits, jnp.float32)
    return -(sum_below + k_rem.astype(jnp.float32) * tval) / jnp.float32(K)
